# baseline (speedup 1.0000x reference)
import functools

import jax
import jax.numpy as jnp
from jax import lax
from jax.experimental import pallas as pl
from jax.experimental.pallas import tpu as pltpu
from jax.experimental.pallas import tpu_sc as plsc

_EMB = 16
_BATCH = 16384
_REGS = 1e-5
_NC = 2
_NS = 16
_NW = _NC * _NS
_BPW = _BATCH // _NW
_GROUPS = _BPW // 16
_ELEMS = _BPW * _EMB
_Q = 4

_TILES = 2_000_000 // 128
_FBLOCK = _TILES * 1024


def _sc_partials(tab_flat, u_idx, p_idx, n_idx):
    mesh = plsc.VectorSubcoreMesh(core_axis_name="c", subcore_axis_name="s")

    @functools.partial(
        pl.kernel,
        mesh=mesh,
        compiler_params=pltpu.CompilerParams(
            needs_layout_passes=False, use_tc_tiling_on_sc=False),
        out_type=jax.ShapeDtypeStruct((_NW, 16), jnp.float32),
        scratch_types=[
            pltpu.VMEM((_BPW,), jnp.int32),
            pltpu.VMEM((_BPW,), jnp.int32),
            pltpu.VMEM((_BPW,), jnp.int32),
            pltpu.VMEM((_ELEMS,), jnp.int32),
            pltpu.VMEM((_ELEMS,), jnp.int32),
            pltpu.VMEM((_ELEMS,), jnp.int32),
            pltpu.VMEM((_ELEMS,), jnp.float32),
            pltpu.VMEM((_ELEMS,), jnp.float32),
            pltpu.VMEM((_ELEMS,), jnp.float32),
            pltpu.VMEM((16,), jnp.float32),
            pltpu.SemaphoreType.DMA,
            pltpu.SemaphoreType.DMA,
            pltpu.SemaphoreType.DMA,
            pltpu.SemaphoreType.DMA,
        ],
    )
    def body(tab, uix, pix, nix, out, ui, pi, ni, uf, pf, nf,
             ue, pe, ne, obuf, sem0, sem1, sem2, sem3):
        wid = lax.axis_index("s") * _NC + lax.axis_index("c")
        sems = (sem0, sem1, sem2, sem3)
        pltpu.sync_copy(uix.at[wid], ui)
        pltpu.sync_copy(pix.at[wid], pi)
        pltpu.sync_copy(nix.at[wid], ni)

        gpq = _GROUPS // _Q
        epq = _ELEMS // _Q

        def build(g, _):
            for sidx, fidx in ((ui, uf), (pi, pf), (ni, nf)):
                rv = sidx[pl.ds(g * 16, 16)]
                base = ((rv >> 7) << 10) + (rv & 127)
                for j in range(_EMB):
                    off = (j & 7) * 128 + (j >> 3) * _FBLOCK
                    fidx[pl.ds(g * 256 + j * 16, 16)] = base + off
            return 0

        for q in range(_Q):
            lax.fori_loop(q * gpq, (q + 1) * gpq, build, 0)
            sl = pl.ds(q * epq, epq)
            pltpu.async_copy(tab.at[uf.at[sl]], ue.at[sl], sems[q])
            pltpu.async_copy(tab.at[pf.at[sl]], pe.at[sl], sems[q])
            pltpu.async_copy(tab.at[nf.at[sl]], ne.at[sl], sems[q])

        lanes = lax.iota(jnp.int32, 16)
        zeros = jnp.zeros((16,), jnp.float32)

        def group(g, carry):
            bpr_acc, sq_acc = carry
            dacc = zeros
            for j in range(_EMB):
                sl = pl.ds(g * 256 + j * 16, 16)
                u = ue[sl]
                p = pe[sl]
                n = ne[sl]
                dacc = dacc + u * (p - n)
                sq_acc = sq_acc + (u * u + (p * p + n * n))
            a = -dacc
            t = jnp.exp(-jnp.abs(a))
            y = 1.0 + t
            z = 0.7 * t
            for _ in range(3):
                z = z - 1.0 + y * jnp.exp(-z)
            bpr_acc = bpr_acc + (jnp.maximum(a, 0.0) + z)
            return bpr_acc, sq_acc

        acc = (zeros, zeros)
        for q in range(_Q):
            sl = pl.ds(q * epq, epq)
            pltpu.make_async_copy(tab.at[uf.at[sl]], ue.at[sl], sems[q]).wait()
            pltpu.make_async_copy(tab.at[pf.at[sl]], pe.at[sl], sems[q]).wait()
            pltpu.make_async_copy(tab.at[nf.at[sl]], ne.at[sl], sems[q]).wait()
            acc = lax.fori_loop(q * gpq, (q + 1) * gpq, group, acc)
        bpr_acc, sq_acc = acc
        bpr_tot = jnp.sum(bpr_acc)
        sq_tot = jnp.sum(sq_acc)
        vals = jnp.where(lanes == 0, bpr_tot,
                         jnp.where(lanes == 1, sq_tot, 0.0))
        obuf[...] = vals
        pltpu.sync_copy(obuf, out.at[wid])

    return body(tab_flat, u_idx, p_idx, n_idx)


def kernel(all_embed, user, pos_item, neg_item):
    tab_flat = (
        all_embed.T.reshape(2, 8, _TILES, 128)
        .transpose(0, 2, 1, 3)
        .reshape(2 * _FBLOCK)
    )
    u2 = user.astype(jnp.int32).reshape(_NW, _BPW)
    p2 = pos_item.astype(jnp.int32).reshape(_NW, _BPW)
    n2 = neg_item.astype(jnp.int32).reshape(_NW, _BPW)
    partials = _sc_partials(tab_flat, u2, p2, n2)
    bpr_loss = jnp.sum(partials[:, 0]) / _BATCH
    reg_loss = _REGS * 0.5 * jnp.sum(partials[:, 1])
    return (bpr_loss, reg_loss)


# speedup vs baseline: 1.0020x; 1.0020x over previous
import functools

import jax
import jax.numpy as jnp
from jax import lax
from jax.experimental import pallas as pl
from jax.experimental.pallas import tpu as pltpu
from jax.experimental.pallas import tpu_sc as plsc

_EMB = 16
_BATCH = 16384
_REGS = 1e-5
_NC = 2
_NS = 16
_NW = _NC * _NS
_BPW = _BATCH // _NW
_GROUPS = _BPW // 16
_ELEMS = _BPW * _EMB
_Q = 2

_TILES = 2_000_000 // 128
_FBLOCK = _TILES * 1024


def _sc_partials(tab_flat, u_idx, p_idx, n_idx):
    mesh = plsc.VectorSubcoreMesh(core_axis_name="c", subcore_axis_name="s")

    @functools.partial(
        pl.kernel,
        mesh=mesh,
        compiler_params=pltpu.CompilerParams(
            needs_layout_passes=False, use_tc_tiling_on_sc=False),
        out_type=jax.ShapeDtypeStruct((_NW, 16), jnp.float32),
        scratch_types=[
            pltpu.VMEM((_BPW,), jnp.int32),
            pltpu.VMEM((_BPW,), jnp.int32),
            pltpu.VMEM((_BPW,), jnp.int32),
            pltpu.VMEM((_ELEMS,), jnp.int32),
            pltpu.VMEM((_ELEMS,), jnp.int32),
            pltpu.VMEM((_ELEMS,), jnp.int32),
            pltpu.VMEM((_ELEMS,), jnp.float32),
            pltpu.VMEM((_ELEMS,), jnp.float32),
            pltpu.VMEM((_ELEMS,), jnp.float32),
            pltpu.VMEM((16,), jnp.float32),
            pltpu.SemaphoreType.DMA,
            pltpu.SemaphoreType.DMA,
            pltpu.SemaphoreType.DMA,
            pltpu.SemaphoreType.DMA,
        ],
    )
    def body(tab, uix, pix, nix, out, ui, pi, ni, uf, pf, nf,
             ue, pe, ne, obuf, sem0, sem1, sem2, sem3):
        wid = lax.axis_index("s") * _NC + lax.axis_index("c")
        sems = (sem0, sem1, sem2, sem3)
        pltpu.sync_copy(uix.at[wid], ui)
        pltpu.sync_copy(pix.at[wid], pi)
        pltpu.sync_copy(nix.at[wid], ni)

        gpq = _GROUPS // _Q
        epq = _ELEMS // _Q

        def build(g, _):
            for sidx, fidx in ((ui, uf), (pi, pf), (ni, nf)):
                rv = sidx[pl.ds(g * 16, 16)]
                base = ((rv >> 7) << 10) + (rv & 127)
                for j in range(_EMB):
                    off = (j & 7) * 128 + (j >> 3) * _FBLOCK
                    fidx[pl.ds(g * 256 + j * 16, 16)] = base + off
            return 0

        for q in range(_Q):
            lax.fori_loop(q * gpq, (q + 1) * gpq, build, 0)
            sl = pl.ds(q * epq, epq)
            pltpu.async_copy(tab.at[uf.at[sl]], ue.at[sl], sems[q])
            pltpu.async_copy(tab.at[pf.at[sl]], pe.at[sl], sems[q])
            pltpu.async_copy(tab.at[nf.at[sl]], ne.at[sl], sems[q])

        lanes = lax.iota(jnp.int32, 16)
        zeros = jnp.zeros((16,), jnp.float32)

        def group(g, carry):
            bpr_acc, sq_acc = carry
            dacc = zeros
            for j in range(_EMB):
                sl = pl.ds(g * 256 + j * 16, 16)
                u = ue[sl]
                p = pe[sl]
                n = ne[sl]
                dacc = dacc + u * (p - n)
                sq_acc = sq_acc + (u * u + (p * p + n * n))
            a = -dacc
            t = jnp.exp(-jnp.abs(a))
            y = 1.0 + t
            z = 0.7 * t
            for _ in range(3):
                z = z - 1.0 + y * jnp.exp(-z)
            bpr_acc = bpr_acc + (jnp.maximum(a, 0.0) + z)
            return bpr_acc, sq_acc

        acc = (zeros, zeros)
        for q in range(_Q):
            sl = pl.ds(q * epq, epq)
            pltpu.make_async_copy(tab.at[uf.at[sl]], ue.at[sl], sems[q]).wait()
            pltpu.make_async_copy(tab.at[pf.at[sl]], pe.at[sl], sems[q]).wait()
            pltpu.make_async_copy(tab.at[nf.at[sl]], ne.at[sl], sems[q]).wait()
            acc = lax.fori_loop(q * gpq, (q + 1) * gpq, group, acc)
        bpr_acc, sq_acc = acc
        bpr_tot = jnp.sum(bpr_acc)
        sq_tot = jnp.sum(sq_acc)
        vals = jnp.where(lanes == 0, bpr_tot,
                         jnp.where(lanes == 1, sq_tot, 0.0))
        obuf[...] = vals
        pltpu.sync_copy(obuf, out.at[wid])

    return body(tab_flat, u_idx, p_idx, n_idx)


def kernel(all_embed, user, pos_item, neg_item):
    tab_flat = (
        all_embed.T.reshape(2, 8, _TILES, 128)
        .transpose(0, 2, 1, 3)
        .reshape(2 * _FBLOCK)
    )
    u2 = user.astype(jnp.int32).reshape(_NW, _BPW)
    p2 = pos_item.astype(jnp.int32).reshape(_NW, _BPW)
    n2 = neg_item.astype(jnp.int32).reshape(_NW, _BPW)
    partials = _sc_partials(tab_flat, u2, p2, n2)
    bpr_loss = jnp.sum(partials[:, 0]) / _BATCH
    reg_loss = _REGS * 0.5 * jnp.sum(partials[:, 1])
    return (bpr_loss, reg_loss)


# speedup vs baseline: 1.0096x; 1.0076x over previous
import functools

import jax
import jax.numpy as jnp
from jax import lax
from jax.experimental import pallas as pl
from jax.experimental.pallas import tpu as pltpu
from jax.experimental.pallas import tpu_sc as plsc

_EMB = 16
_BATCH = 16384
_REGS = 1e-5
_NC = 2
_NS = 16
_NW = _NC * _NS
_BPW = _BATCH // _NW
_GROUPS = _BPW // 16
_ELEMS = _BPW * _EMB
_Q = 1

_TILES = 2_000_000 // 128
_FBLOCK = _TILES * 1024


def _sc_partials(tab_flat, u_idx, p_idx, n_idx):
    mesh = plsc.VectorSubcoreMesh(core_axis_name="c", subcore_axis_name="s")

    @functools.partial(
        pl.kernel,
        mesh=mesh,
        compiler_params=pltpu.CompilerParams(
            needs_layout_passes=False, use_tc_tiling_on_sc=False),
        out_type=jax.ShapeDtypeStruct((_NW, 16), jnp.float32),
        scratch_types=[
            pltpu.VMEM((_BPW,), jnp.int32),
            pltpu.VMEM((_BPW,), jnp.int32),
            pltpu.VMEM((_BPW,), jnp.int32),
            pltpu.VMEM((_ELEMS,), jnp.int32),
            pltpu.VMEM((_ELEMS,), jnp.int32),
            pltpu.VMEM((_ELEMS,), jnp.int32),
            pltpu.VMEM((_ELEMS,), jnp.float32),
            pltpu.VMEM((_ELEMS,), jnp.float32),
            pltpu.VMEM((_ELEMS,), jnp.float32),
            pltpu.VMEM((16,), jnp.float32),
            pltpu.SemaphoreType.DMA,
            pltpu.SemaphoreType.DMA,
            pltpu.SemaphoreType.DMA,
            pltpu.SemaphoreType.DMA,
        ],
    )
    def body(tab, uix, pix, nix, out, ui, pi, ni, uf, pf, nf,
             ue, pe, ne, obuf, sem0, sem1, sem2, sem3):
        wid = lax.axis_index("s") * _NC + lax.axis_index("c")
        sems = (sem0, sem1, sem2, sem3)
        pltpu.sync_copy(uix.at[wid], ui)
        pltpu.sync_copy(pix.at[wid], pi)
        pltpu.sync_copy(nix.at[wid], ni)

        gpq = _GROUPS // _Q
        epq = _ELEMS // _Q

        def build(g, _):
            for sidx, fidx in ((ui, uf), (pi, pf), (ni, nf)):
                rv = sidx[pl.ds(g * 16, 16)]
                base = ((rv >> 7) << 10) + (rv & 127)
                for j in range(_EMB):
                    off = (j & 7) * 128 + (j >> 3) * _FBLOCK
                    fidx[pl.ds(g * 256 + j * 16, 16)] = base + off
            return 0

        for q in range(_Q):
            lax.fori_loop(q * gpq, (q + 1) * gpq, build, 0)
            sl = pl.ds(q * epq, epq)
            pltpu.async_copy(tab.at[uf.at[sl]], ue.at[sl], sems[q])
            pltpu.async_copy(tab.at[pf.at[sl]], pe.at[sl], sems[q])
            pltpu.async_copy(tab.at[nf.at[sl]], ne.at[sl], sems[q])

        lanes = lax.iota(jnp.int32, 16)
        zeros = jnp.zeros((16,), jnp.float32)

        def group(g, carry):
            bpr_acc, sq_acc = carry
            dacc = zeros
            for j in range(_EMB):
                sl = pl.ds(g * 256 + j * 16, 16)
                u = ue[sl]
                p = pe[sl]
                n = ne[sl]
                dacc = dacc + u * (p - n)
                sq_acc = sq_acc + (u * u + (p * p + n * n))
            a = -dacc
            t = jnp.exp(-jnp.abs(a))
            y = 1.0 + t
            z = 0.7 * t
            for _ in range(3):
                z = z - 1.0 + y * jnp.exp(-z)
            bpr_acc = bpr_acc + (jnp.maximum(a, 0.0) + z)
            return bpr_acc, sq_acc

        acc = (zeros, zeros)
        for q in range(_Q):
            sl = pl.ds(q * epq, epq)
            pltpu.make_async_copy(tab.at[uf.at[sl]], ue.at[sl], sems[q]).wait()
            pltpu.make_async_copy(tab.at[pf.at[sl]], pe.at[sl], sems[q]).wait()
            pltpu.make_async_copy(tab.at[nf.at[sl]], ne.at[sl], sems[q]).wait()
            acc = lax.fori_loop(q * gpq, (q + 1) * gpq, group, acc)
        bpr_acc, sq_acc = acc
        bpr_tot = jnp.sum(bpr_acc)
        sq_tot = jnp.sum(sq_acc)
        vals = jnp.where(lanes == 0, bpr_tot,
                         jnp.where(lanes == 1, sq_tot, 0.0))
        obuf[...] = vals
        pltpu.sync_copy(obuf, out.at[wid])

    return body(tab_flat, u_idx, p_idx, n_idx)


def kernel(all_embed, user, pos_item, neg_item):
    tab_flat = (
        all_embed.T.reshape(2, 8, _TILES, 128)
        .transpose(0, 2, 1, 3)
        .reshape(2 * _FBLOCK)
    )
    u2 = user.astype(jnp.int32).reshape(_NW, _BPW)
    p2 = pos_item.astype(jnp.int32).reshape(_NW, _BPW)
    n2 = neg_item.astype(jnp.int32).reshape(_NW, _BPW)
    partials = _sc_partials(tab_flat, u2, p2, n2)
    bpr_loss = jnp.sum(partials[:, 0]) / _BATCH
    reg_loss = _REGS * 0.5 * jnp.sum(partials[:, 1])
    return (bpr_loss, reg_loss)
